# Initial kernel scaffold; baseline (speedup 1.0000x reference)
import jax
import jax.numpy as jnp
from jax.experimental import pallas as pl


def kernel(x, table):
    raise NotImplementedError("write your pallas kernel here")


# speedup vs baseline: 8.9642x; 8.9642x over previous
import functools

import jax
import jax.numpy as jnp
from jax import lax
from jax.experimental import pallas as pl
from jax.experimental.pallas import tpu as pltpu
from jax.experimental.pallas import tpu_sc as plsc

D = 128
NC = 2
NS = 16
NW = NC * NS
C = 256
K = C // 128
SUPER = 1024
CPS = SUPER // C
NBUF = 2


def _sc_gather(table, idx2d, total_rows):
    rows_per_w = total_rows // NW
    n_super = rows_per_w // SUPER

    mesh = plsc.VectorSubcoreMesh(core_axis_name="c", subcore_axis_name="s")

    @functools.partial(
        pl.kernel,
        mesh=mesh,
        out_type=jax.ShapeDtypeStruct((total_rows, D), jnp.float32),
        scratch_types=[
            pltpu.VMEM((SUPER // 128, 128), jnp.int32),
            pltpu.VMEM((NBUF, C, D), jnp.float32),
            pltpu.SemaphoreType.DMA,
            pltpu.SemaphoreType.DMA,
            pltpu.SemaphoreType.DMA,
        ],
    )
    def k(table_hbm, idx_hbm, out_hbm, idx_v, rows_v, gsem, osem0, osem1):
        wid = lax.axis_index("s") * NC + lax.axis_index("c")
        base = wid * rows_per_w
        osems = [osem0, osem1]

        def body(ss, carry):
            sbase = base + ss * SUPER
            row0 = pl.multiple_of(sbase // 128, SUPER // 128)
            pltpu.sync_copy(idx_hbm.at[pl.ds(row0, SUPER // 128)], idx_v)
            for c in range(CPS):
                b = c % NBUF
                off = sbase + c * C

                def drain():
                    pltpu.make_async_copy(
                        rows_v.at[b],
                        out_hbm.at[pl.ds(off - NBUF * C, C)],
                        osems[b],
                    ).wait()

                if c < NBUF:
                    pl.when(ss > 0)(drain)
                else:
                    drain()

                cps = [
                    pltpu.async_copy(
                        table_hbm.at[idx_v.at[c * K + j]],
                        rows_v.at[b, pl.ds(j * 128, 128)],
                        gsem,
                    )
                    for j in range(K)
                ]
                for cp in cps:
                    cp.wait()
                pltpu.async_copy(
                    rows_v.at[b], out_hbm.at[pl.ds(off, C)], osems[b]
                )
            return carry

        lax.fori_loop(0, n_super, body, 0)

        for b in range(NBUF):
            off = base + rows_per_w - (NBUF - b) * C
            pltpu.make_async_copy(
                rows_v.at[b], out_hbm.at[pl.ds(off, C)], osems[b]
            ).wait()

    return k(table, idx2d)


def kernel(x, table):
    batch, seq = x.shape
    total_rows = batch * seq
    idx2d = x.reshape(total_rows // 128, 128).astype(jnp.int32)
    out = _sc_gather(table, idx2d, total_rows)
    return out.reshape(batch, seq, D)


# speedup vs baseline: 9.2307x; 1.0297x over previous
import functools

import jax
import jax.numpy as jnp
from jax import lax
from jax.experimental import pallas as pl
from jax.experimental.pallas import tpu as pltpu
from jax.experimental.pallas import tpu_sc as plsc

D = 128
NC = 2
NS = 16
NW = NC * NS
C = 256
K = C // 128
SUPER = 1024
CPS = SUPER // C
NBUF = 2


def _sc_gather(table, idx2d, total_rows):
    rows_per_w = total_rows // NW
    n_super = rows_per_w // SUPER

    mesh = plsc.VectorSubcoreMesh(core_axis_name="c", subcore_axis_name="s")

    @functools.partial(
        pl.kernel,
        mesh=mesh,
        out_type=jax.ShapeDtypeStruct((total_rows, D), jnp.float32),
        scratch_types=[
            pltpu.VMEM((2, SUPER // 128, 128), jnp.int32),
            pltpu.VMEM((NBUF, C, D), jnp.float32),
            pltpu.SemaphoreType.DMA,
            pltpu.SemaphoreType.DMA,
            pltpu.SemaphoreType.DMA,
            pltpu.SemaphoreType.DMA,
        ],
    )
    def k(table_hbm, idx_hbm, out_hbm, idx_v, rows_v, isem, gsem, osem0, osem1):
        wid = lax.axis_index("s") * NC + lax.axis_index("c")
        base = wid * rows_per_w
        osems = [osem0, osem1]

        def idx_rows(ss):
            return pl.multiple_of((base + ss * SUPER) // 128, SUPER // 128)

        pltpu.async_copy(
            idx_hbm.at[pl.ds(idx_rows(0), SUPER // 128)], idx_v.at[0], isem
        )

        def body(ss, carry):
            ib = ss % 2
            sbase = base + ss * SUPER
            pltpu.make_async_copy(
                idx_hbm.at[pl.ds(idx_rows(ss), SUPER // 128)],
                idx_v.at[ib],
                isem,
            ).wait()

            @pl.when(ss + 1 < n_super)
            def _():
                pltpu.async_copy(
                    idx_hbm.at[pl.ds(idx_rows(ss + 1), SUPER // 128)],
                    idx_v.at[(ss + 1) % 2],
                    isem,
                )

            for c in range(CPS):
                b = c % NBUF
                off = sbase + c * C

                def drain():
                    pltpu.make_async_copy(
                        rows_v.at[b],
                        out_hbm.at[pl.ds(off - NBUF * C, C)],
                        osems[b],
                    ).wait()

                if c < NBUF:
                    pl.when(ss > 0)(drain)
                else:
                    drain()

                cps = [
                    pltpu.async_copy(
                        table_hbm.at[idx_v.at[ib, c * K + j]],
                        rows_v.at[b, pl.ds(j * 128, 128)],
                        gsem,
                    )
                    for j in range(K)
                ]
                for cp in cps:
                    cp.wait()
                pltpu.async_copy(
                    rows_v.at[b], out_hbm.at[pl.ds(off, C)], osems[b]
                )
            return carry

        lax.fori_loop(0, n_super, body, 0)

        for b in range(NBUF):
            off = base + rows_per_w - (NBUF - b) * C
            pltpu.make_async_copy(
                rows_v.at[b], out_hbm.at[pl.ds(off, C)], osems[b]
            ).wait()

    return k(table, idx2d)


def kernel(x, table):
    batch, seq = x.shape
    total_rows = batch * seq
    idx2d = x.reshape(total_rows // 128, 128).astype(jnp.int32)
    out = _sc_gather(table, idx2d, total_rows)
    return out.reshape(batch, seq, D)
